# baseline (speedup 1.0000x reference)
import functools

import jax
import jax.numpy as jnp
from jax import lax
from jax.experimental import pallas as pl
from jax.experimental.pallas import tpu as pltpu
from jax.experimental.pallas import tpu_sc as plsc

N = 10000
E = 320000
CLUST = 100

NC = 2
NS = 16
NW = NC * NS
K = 80
NB = 125
EPAD = NW * NB * K
ROWS_PT = 640
NPAD = NS * ROWS_PT

_SC_MESH = plsc.VectorSubcoreMesh(
    core_axis_name="c", subcore_axis_name="s", num_cores=NC, num_subcores=NS)


def _sc_degree(dst3):
    @functools.partial(
        pl.kernel,
        out_type=jax.ShapeDtypeStruct((NC, NPAD), jnp.float32),
        mesh=_SC_MESH,
        scratch_types=[
            pltpu.VMEM((NB, K), jnp.int32),
            pltpu.VMEM((K,), jnp.float32),
            pltpu.VMEM((ROWS_PT,), jnp.float32),
            pltpu.VMEM_SHARED((NPAD,), jnp.float32),
        ],
    )
    def deg_kernel(dst_hbm, out_hbm, dst_v, ones_v, z_v, acc):
        c = lax.axis_index("c")
        s = lax.axis_index("s")
        wid = c * NS + s
        r0 = s * ROWS_PT
        for k in range(K // 16):
            ones_v[pl.ds(k * 16, 16)] = jnp.ones((16,), jnp.float32)
        for k in range(ROWS_PT // 16):
            z_v[pl.ds(k * 16, 16)] = jnp.zeros((16,), jnp.float32)
        pltpu.sync_copy(z_v, acc.at[pl.ds(r0, ROWS_PT)])
        pltpu.sync_copy(dst_hbm.at[wid], dst_v)
        plsc.subcore_barrier()

        def body(j, carry):
            pltpu.sync_copy(ones_v, acc.at[dst_v.at[j]], add=True)
            return carry

        lax.fori_loop(0, NB, body, 0)
        plsc.subcore_barrier()
        pltpu.sync_copy(acc.at[pl.ds(r0, ROWS_PT)],
                        out_hbm.at[c, pl.ds(r0, ROWS_PT)])

    return deg_kernel(dst3)


def _sc_aggregate(xs, src3, dst3, zs, F):
    @functools.partial(
        pl.kernel,
        out_type=jax.ShapeDtypeStruct((NC, NPAD, F), jnp.float32),
        mesh=_SC_MESH,
        scratch_types=[
            pltpu.VMEM((4, K), jnp.int32),
            pltpu.VMEM((4, K), jnp.int32),
            pltpu.VMEM((3, K, F), jnp.float32),
            pltpu.VMEM_SHARED((NPAD, F), jnp.float32),
            pltpu.SemaphoreType.DMA((2,)),
            pltpu.SemaphoreType.DMA((2,)),
            pltpu.SemaphoreType.DMA,
        ],
    )
    def agg_kernel(xs_hbm, src_hbm, dst_hbm, zs_hbm, out_hbm,
                   isrc, idst, rows_v, acc, isem, gsem, ssem):
        c = lax.axis_index("c")
        s = lax.axis_index("s")
        wid = c * NS + s
        r0 = s * ROWS_PT
        pltpu.sync_copy(zs_hbm.at[pl.ds(r0, ROWS_PT)],
                        acc.at[pl.ds(r0, ROWS_PT)])
        pltpu.sync_copy(src_hbm.at[wid, 0], isrc.at[0])
        pltpu.sync_copy(dst_hbm.at[wid, 0], idst.at[0])
        pltpu.sync_copy(src_hbm.at[wid, 1], isrc.at[1])
        pltpu.sync_copy(dst_hbm.at[wid, 1], idst.at[1])
        pltpu.async_copy(src_hbm.at[wid, 2], isrc.at[2], isem.at[0])
        pltpu.async_copy(dst_hbm.at[wid, 2], idst.at[2], isem.at[0])
        plsc.subcore_barrier()
        pltpu.async_copy(xs_hbm.at[isrc.at[0]], rows_v.at[0], gsem.at[0])
        pltpu.async_copy(xs_hbm.at[isrc.at[1]], rows_v.at[1], gsem.at[1])

        def body(j, carry):
            m = lax.rem(j, 3)
            m2 = lax.rem(j + 2, 3)
            hj = lax.rem(j, 2)
            h2 = hj
            i_cur = lax.rem(j, 4)
            i_nxt2 = lax.rem(j + 2, 4)
            i_nxt3 = lax.rem(j + 3, 4)
            pltpu.make_async_copy(xs_hbm.at[isrc.at[i_cur]],
                                  rows_v.at[m], gsem.at[hj]).wait()

            @pl.when(j >= 1)
            def _():
                pltpu.make_async_copy(rows_v.at[m2],
                                      acc.at[idst.at[i_nxt3]], ssem).wait()

            @pl.when(j + 3 < NB)
            def _():
                pltpu.async_copy(src_hbm.at[wid, j + 3], isrc.at[i_nxt3],
                                 isem.at[1 - hj])
                pltpu.async_copy(dst_hbm.at[wid, j + 3], idst.at[i_nxt3],
                                 isem.at[1 - hj])

            @pl.when(j + 2 < NB)
            def _():
                pltpu.make_async_copy(src_hbm.at[wid, j + 2],
                                      isrc.at[i_nxt2], isem.at[h2]).wait()
                pltpu.make_async_copy(dst_hbm.at[wid, j + 2],
                                      idst.at[i_nxt2], isem.at[h2]).wait()
                pltpu.async_copy(xs_hbm.at[isrc.at[i_nxt2]],
                                 rows_v.at[m2], gsem.at[h2])

            pltpu.async_copy(rows_v.at[m], acc.at[idst.at[i_cur]],
                             ssem, add=True)
            return carry

        lax.fori_loop(0, NB, body, 0)
        pltpu.make_async_copy(rows_v.at[(NB - 1) % 3],
                              acc.at[idst.at[(NB - 1) % 4]], ssem).wait()
        plsc.subcore_barrier()
        pltpu.sync_copy(acc.at[pl.ds(r0, ROWS_PT)],
                        out_hbm.at[c, pl.ds(r0, ROWS_PT)])

    return agg_kernel(xs, src3, dst3, zs)


def _elu(v):
    return jnp.where(v > 0, v, jnp.exp(jnp.minimum(v, 0.0)) - 1.0)


def _tc_first(x, W, degp, B=2000):
    F = W.shape[1]

    def body(x_ref, w_ref, d_ref, xs_ref, dinv_ref):
        deg = d_ref[0] + d_ref[1] + 1.0
        dinv = lax.rsqrt(deg)
        dinv_ref[...] = dinv
        xs_ref[...] = dinv * jnp.dot(x_ref[...], w_ref[...],
                                     preferred_element_type=jnp.float32)

    grid = N // B
    return pl.pallas_call(
        body,
        grid=(grid,),
        in_specs=[
            pl.BlockSpec((B, x.shape[1]), lambda i: (i, 0)),
            pl.BlockSpec(W.shape, lambda i: (0, 0)),
            pl.BlockSpec((NC, B, 1), lambda i: (0, i, 0)),
        ],
        out_specs=[
            pl.BlockSpec((B, F), lambda i: (i, 0)),
            pl.BlockSpec((B, 1), lambda i: (i, 0)),
        ],
        out_shape=[
            jax.ShapeDtypeStruct((N, F), jnp.float32),
            jax.ShapeDtypeStruct((N, 1), jnp.float32),
        ],
    )(x, W, degp)


def _tc_mid(part, xs, dinv, b, W, apply_elu, B=2000):
    Fin = xs.shape[1]
    F = W.shape[1]

    def body(p_ref, xs_ref, d_ref, b_ref, w_ref, o_ref):
        dinv = d_ref[...]
        h = dinv * (p_ref[0] + p_ref[1] + xs_ref[...]) + b_ref[...]
        if apply_elu:
            h = _elu(h)
        o_ref[...] = dinv * jnp.dot(h, w_ref[...],
                                    preferred_element_type=jnp.float32)

    grid = N // B
    return pl.pallas_call(
        body,
        grid=(grid,),
        in_specs=[
            pl.BlockSpec((NC, B, Fin), lambda i: (0, i, 0)),
            pl.BlockSpec((B, Fin), lambda i: (i, 0)),
            pl.BlockSpec((B, 1), lambda i: (i, 0)),
            pl.BlockSpec((1, Fin), lambda i: (0, 0)),
            pl.BlockSpec(W.shape, lambda i: (0, 0)),
        ],
        out_specs=pl.BlockSpec((B, F), lambda i: (i, 0)),
        out_shape=jax.ShapeDtypeStruct((N, F), jnp.float32),
    )(part, xs, dinv, b, W)


def _tc_bn_stats(part, xs, dinv, b, B=2000):
    F = xs.shape[1]

    def body(p_ref, xs_ref, d_ref, b_ref, h_ref, st_ref):
        i = pl.program_id(0)
        h = d_ref[...] * (p_ref[0] + p_ref[1] + xs_ref[...]) + b_ref[...]
        h_ref[...] = h
        st = jnp.concatenate(
            [jnp.sum(h, 0, keepdims=True),
             jnp.sum(h * h, 0, keepdims=True)], 0)

        @pl.when(i == 0)
        def _():
            st_ref[...] = st

        @pl.when(i > 0)
        def _():
            st_ref[...] += st

    grid = N // B
    return pl.pallas_call(
        body,
        grid=(grid,),
        in_specs=[
            pl.BlockSpec((NC, B, F), lambda i: (0, i, 0)),
            pl.BlockSpec((B, F), lambda i: (i, 0)),
            pl.BlockSpec((B, 1), lambda i: (i, 0)),
            pl.BlockSpec((1, F), lambda i: (0, 0)),
        ],
        out_specs=[
            pl.BlockSpec((B, F), lambda i: (i, 0)),
            pl.BlockSpec((2, F), lambda i: (0, 0)),
        ],
        out_shape=[
            jax.ShapeDtypeStruct((N, F), jnp.float32),
            jax.ShapeDtypeStruct((2, F), jnp.float32),
        ],
    )(part, xs, dinv, b)


def _tc_bn_apply(h3, stats, gamma, beta, dinv, W, B=2000):
    F = h3.shape[1]
    Fo = W.shape[1]

    def body(h_ref, st_ref, g_ref, be_ref, d_ref, w_ref, o_ref):
        mean = st_ref[0:1] * (1.0 / N)
        var = st_ref[1:2] * (1.0 / N) - mean * mean
        y = (h_ref[...] - mean) * lax.rsqrt(var + 1e-5) * g_ref[...] + be_ref[...]
        o_ref[...] = d_ref[...] * jnp.dot(y, w_ref[...],
                                          preferred_element_type=jnp.float32)

    grid = N // B
    return pl.pallas_call(
        body,
        grid=(grid,),
        in_specs=[
            pl.BlockSpec((B, F), lambda i: (i, 0)),
            pl.BlockSpec((2, F), lambda i: (0, 0)),
            pl.BlockSpec((1, F), lambda i: (0, 0)),
            pl.BlockSpec((1, F), lambda i: (0, 0)),
            pl.BlockSpec((B, 1), lambda i: (i, 0)),
            pl.BlockSpec(W.shape, lambda i: (0, 0)),
        ],
        out_specs=pl.BlockSpec((B, Fo), lambda i: (i, 0)),
        out_shape=jax.ShapeDtypeStruct((N, Fo), jnp.float32),
    )(h3, stats, gamma, beta, dinv, W)


def _tc_head(part, xs, dinv, b, fc1_Wt, fc1_b, fc2_Wt, fc2_b, fc3_Wt, fc3_b,
             B=2000):
    Fin = xs.shape[1]
    G = fc1_Wt.shape[1]
    H = fc2_Wt.shape[1]
    Fo = fc3_Wt.shape[1]
    NG_ = G // CLUST

    def body(p_ref, xs_ref, d_ref, b_ref, w1_ref, b1_ref, w2_ref, b2_ref,
             w3_ref, b3_ref, o_ref):
        z = d_ref[...] * (p_ref[0] + p_ref[1] + xs_ref[...]) + b_ref[...]
        t = _elu(jnp.dot(z, w1_ref[...],
                         preferred_element_type=jnp.float32) + b1_ref[...])
        w2 = w2_ref[...]
        w3 = w3_ref[...]
        b2 = b2_ref[...]
        b3 = b3_ref[...]
        vs = []
        for j in range(NG_):
            u = jnp.dot(t[:, j * CLUST:(j + 1) * CLUST], w2,
                        preferred_element_type=jnp.float32) + b2
            vs.append(jnp.dot(u, w3, preferred_element_type=jnp.float32) + b3)
        o_ref[...] = jnp.stack(vs, axis=1).reshape(B * NG_, Fo)

    grid = N // B
    return pl.pallas_call(
        body,
        grid=(grid,),
        in_specs=[
            pl.BlockSpec((NC, B, Fin), lambda i: (0, i, 0)),
            pl.BlockSpec((B, Fin), lambda i: (i, 0)),
            pl.BlockSpec((B, 1), lambda i: (i, 0)),
            pl.BlockSpec((1, Fin), lambda i: (0, 0)),
            pl.BlockSpec(fc1_Wt.shape, lambda i: (0, 0)),
            pl.BlockSpec((1, G), lambda i: (0, 0)),
            pl.BlockSpec(fc2_Wt.shape, lambda i: (0, 0)),
            pl.BlockSpec((1, H), lambda i: (0, 0)),
            pl.BlockSpec(fc3_Wt.shape, lambda i: (0, 0)),
            pl.BlockSpec((1, Fo), lambda i: (0, 0)),
        ],
        out_specs=pl.BlockSpec((B * NG_, Fo), lambda i: (i, 0)),
        out_shape=jax.ShapeDtypeStruct((N * NG_, Fo), jnp.float32),
    )(part, xs, dinv, b, fc1_Wt, fc1_b, fc2_Wt, fc2_b, fc3_Wt, fc3_b)


def kernel(x, adj, num_graphs, in_batch, cluster, W1, b1, W2, b2, W3, b3,
           bn_gamma, bn_beta, W4, b4, W5, b5, fc1_W, fc1_b, fc2_W, fc2_b,
           fc3_W, fc3_b):
    if EPAD == E:
        src3 = adj[0].reshape(NW, NB, K)
        dst3 = adj[1].reshape(NW, NB, K)
    else:
        ar = jnp.arange(EPAD - E, dtype=jnp.int32)
        src3 = jnp.concatenate([adj[0], ar % N]).reshape(NW, NB, K)
        dst3 = jnp.concatenate([adj[1], N + ar % (NPAD - N)]).reshape(NW, NB, K)
    zs128 = jnp.zeros((NPAD, 128), jnp.float32)

    W4p = jnp.pad(W4, ((0, 0), (0, 64)))
    b4p = jnp.pad(b4, (0, 64))
    W5p = jnp.pad(W5, ((0, 64), (0, 64)))
    b5p = jnp.pad(b5, (0, 64))
    fc1_Wtp = jnp.pad(fc1_W.T, ((0, 64), (0, 0)))

    degp = _sc_degree(dst3).reshape(NC, NPAD, 1)

    xs1, dinv = _tc_first(x, W1, degp)
    p1 = _sc_aggregate(xs1, src3, dst3, zs128, 128)
    xs2 = _tc_mid(p1, xs1, dinv, b1.reshape(1, -1), W2, True)
    p2 = _sc_aggregate(xs2, src3, dst3, zs128, 128)
    xs3 = _tc_mid(p2, xs2, dinv, b2.reshape(1, -1), W3, True)
    p3 = _sc_aggregate(xs3, src3, dst3, zs128, 128)
    h3, stats = _tc_bn_stats(p3, xs3, dinv, b3.reshape(1, -1))
    xs4 = _tc_bn_apply(h3, stats, bn_gamma.reshape(1, -1),
                       bn_beta.reshape(1, -1), dinv, W4p)
    p4 = _sc_aggregate(xs4, src3, dst3, zs128, 128)
    xs5 = _tc_mid(p4, xs4, dinv, b4p.reshape(1, -1), W5p, False)
    p5 = _sc_aggregate(xs5, src3, dst3, zs128, 128)
    return _tc_head(p5, xs5, dinv, b5p.reshape(1, -1), fc1_Wtp,
                    fc1_b.reshape(1, -1), fc2_W.T, fc2_b.reshape(1, -1),
                    fc3_W.T, fc3_b.reshape(1, -1))


# speedup vs baseline: 1.0218x; 1.0218x over previous
import functools

import jax
import jax.numpy as jnp
from jax import lax
from jax.experimental import pallas as pl
from jax.experimental.pallas import tpu as pltpu
from jax.experimental.pallas import tpu_sc as plsc

N = 10000
E = 320000
CLUST = 100

NC = 2
NS = 16
NW = NC * NS
K = 80
NB = 125
EPAD = NW * NB * K
ROWS_PT = 640
NPAD = NS * ROWS_PT

_SC_MESH = plsc.VectorSubcoreMesh(
    core_axis_name="c", subcore_axis_name="s", num_cores=NC, num_subcores=NS)


def _sc_degree(dst3):
    @functools.partial(
        pl.kernel,
        out_type=jax.ShapeDtypeStruct((NC, NPAD), jnp.float32),
        mesh=_SC_MESH,
        scratch_types=[
            pltpu.VMEM((NB, K), jnp.int32),
            pltpu.VMEM((K,), jnp.float32),
            pltpu.VMEM((ROWS_PT,), jnp.float32),
            pltpu.VMEM_SHARED((NPAD,), jnp.float32),
        ],
    )
    def deg_kernel(dst_hbm, out_hbm, dst_v, ones_v, z_v, acc):
        c = lax.axis_index("c")
        s = lax.axis_index("s")
        wid = c * NS + s
        r0 = s * ROWS_PT
        for k in range(K // 16):
            ones_v[pl.ds(k * 16, 16)] = jnp.ones((16,), jnp.float32)
        for k in range(ROWS_PT // 16):
            z_v[pl.ds(k * 16, 16)] = jnp.zeros((16,), jnp.float32)
        pltpu.sync_copy(z_v, acc.at[pl.ds(r0, ROWS_PT)])
        pltpu.sync_copy(dst_hbm.at[wid], dst_v)
        plsc.subcore_barrier()

        def body(j, carry):
            pltpu.sync_copy(ones_v, acc.at[dst_v.at[j]], add=True)
            return carry

        lax.fori_loop(0, NB, body, 0)
        plsc.subcore_barrier()
        pltpu.sync_copy(acc.at[pl.ds(r0, ROWS_PT)],
                        out_hbm.at[c, pl.ds(r0, ROWS_PT)])

    return deg_kernel(dst3)


def _sc_aggregate(xs, src3, dst3, zs, F):
    @functools.partial(
        pl.kernel,
        out_type=jax.ShapeDtypeStruct((NC, NPAD, F), jnp.float32),
        mesh=_SC_MESH,
        scratch_types=[
            pltpu.VMEM((4, K), jnp.int32),
            pltpu.VMEM((4, K), jnp.int32),
            pltpu.VMEM((3, K, F), jnp.float32),
            pltpu.VMEM_SHARED((NPAD, F), jnp.float32),
            pltpu.SemaphoreType.DMA((2,)),
            pltpu.SemaphoreType.DMA((2,)),
            pltpu.SemaphoreType.DMA,
        ],
    )
    def agg_kernel(xs_hbm, src_hbm, dst_hbm, zs_hbm, out_hbm,
                   isrc, idst, rows_v, acc, isem, gsem, ssem):
        c = lax.axis_index("c")
        s = lax.axis_index("s")
        wid = c * NS + s
        r0 = s * ROWS_PT
        pltpu.sync_copy(zs_hbm.at[pl.ds(r0, ROWS_PT)],
                        acc.at[pl.ds(r0, ROWS_PT)])
        pltpu.sync_copy(src_hbm.at[wid, 0], isrc.at[0])
        pltpu.sync_copy(dst_hbm.at[wid, 0], idst.at[0])
        pltpu.sync_copy(src_hbm.at[wid, 1], isrc.at[1])
        pltpu.sync_copy(dst_hbm.at[wid, 1], idst.at[1])
        pltpu.async_copy(src_hbm.at[wid, 2], isrc.at[2], isem.at[0])
        pltpu.async_copy(dst_hbm.at[wid, 2], idst.at[2], isem.at[0])
        plsc.subcore_barrier()
        pltpu.async_copy(xs_hbm.at[isrc.at[0]], rows_v.at[0], gsem.at[0])
        pltpu.async_copy(xs_hbm.at[isrc.at[1]], rows_v.at[1], gsem.at[1])

        def body(j, carry):
            m = lax.rem(j, 3)
            m2 = lax.rem(j + 2, 3)
            hj = lax.rem(j, 2)
            h2 = hj
            i_cur = lax.rem(j, 4)
            i_nxt2 = lax.rem(j + 2, 4)
            i_nxt3 = lax.rem(j + 3, 4)
            pltpu.make_async_copy(xs_hbm.at[isrc.at[i_cur]],
                                  rows_v.at[m], gsem.at[hj]).wait()

            @pl.when(j >= 1)
            def _():
                pltpu.make_async_copy(rows_v.at[m2],
                                      acc.at[idst.at[i_nxt3]], ssem).wait()

            @pl.when(j + 3 < NB)
            def _():
                pltpu.async_copy(src_hbm.at[wid, j + 3], isrc.at[i_nxt3],
                                 isem.at[1 - hj])
                pltpu.async_copy(dst_hbm.at[wid, j + 3], idst.at[i_nxt3],
                                 isem.at[1 - hj])

            @pl.when(j + 2 < NB)
            def _():
                pltpu.make_async_copy(src_hbm.at[wid, j + 2],
                                      isrc.at[i_nxt2], isem.at[h2]).wait()
                pltpu.make_async_copy(dst_hbm.at[wid, j + 2],
                                      idst.at[i_nxt2], isem.at[h2]).wait()
                pltpu.async_copy(xs_hbm.at[isrc.at[i_nxt2]],
                                 rows_v.at[m2], gsem.at[h2])

            pltpu.async_copy(rows_v.at[m], acc.at[idst.at[i_cur]],
                             ssem, add=True)
            return carry

        lax.fori_loop(0, NB, body, 0)
        pltpu.make_async_copy(rows_v.at[(NB - 1) % 3],
                              acc.at[idst.at[(NB - 1) % 4]], ssem).wait()
        plsc.subcore_barrier()
        pltpu.sync_copy(acc.at[pl.ds(r0, ROWS_PT)],
                        out_hbm.at[c, pl.ds(r0, ROWS_PT)])

    return agg_kernel(xs, src3, dst3, zs)


def _elu(v):
    return jnp.where(v > 0, v, jnp.exp(jnp.minimum(v, 0.0)) - 1.0)


def _tc_first(x, W, degp, B=2000):
    F = W.shape[1]

    def body(x_ref, w_ref, d_ref, xs_ref, dinv_ref):
        deg = d_ref[0] + d_ref[1] + 1.0
        dinv = lax.rsqrt(deg)
        dinv_ref[...] = dinv
        xs_ref[...] = dinv * jnp.dot(x_ref[...], w_ref[...],
                                     preferred_element_type=jnp.float32)

    grid = N // B
    return pl.pallas_call(
        body,
        grid=(grid,),
        in_specs=[
            pl.BlockSpec((B, x.shape[1]), lambda i: (i, 0)),
            pl.BlockSpec(W.shape, lambda i: (0, 0)),
            pl.BlockSpec((NC, B, 1), lambda i: (0, i, 0)),
        ],
        out_specs=[
            pl.BlockSpec((B, F), lambda i: (i, 0)),
            pl.BlockSpec((B, 1), lambda i: (i, 0)),
        ],
        out_shape=[
            jax.ShapeDtypeStruct((N, F), jnp.float32),
            jax.ShapeDtypeStruct((N, 1), jnp.float32),
        ],
    )(x, W, degp)


def _tc_mid(part, xs, dinv, b, W, apply_elu, B=2000):
    Fin = xs.shape[1]
    F = W.shape[1]

    def body(p_ref, xs_ref, d_ref, b_ref, w_ref, o_ref):
        dinv = d_ref[...]
        h = dinv * (p_ref[0] + p_ref[1] + xs_ref[...]) + b_ref[...]
        if apply_elu:
            h = _elu(h)
        o_ref[...] = dinv * jnp.dot(h, w_ref[...],
                                    preferred_element_type=jnp.float32)

    grid = N // B
    return pl.pallas_call(
        body,
        grid=(grid,),
        in_specs=[
            pl.BlockSpec((NC, B, Fin), lambda i: (0, i, 0)),
            pl.BlockSpec((B, Fin), lambda i: (i, 0)),
            pl.BlockSpec((B, 1), lambda i: (i, 0)),
            pl.BlockSpec((1, Fin), lambda i: (0, 0)),
            pl.BlockSpec(W.shape, lambda i: (0, 0)),
        ],
        out_specs=pl.BlockSpec((B, F), lambda i: (i, 0)),
        out_shape=jax.ShapeDtypeStruct((N, F), jnp.float32),
    )(part, xs, dinv, b, W)


def _tc_bn_stats(part, xs, dinv, b, B=2000):
    F = xs.shape[1]

    def body(p_ref, xs_ref, d_ref, b_ref, h_ref, st_ref):
        i = pl.program_id(0)
        h = d_ref[...] * (p_ref[0] + p_ref[1] + xs_ref[...]) + b_ref[...]
        h_ref[...] = h
        st = jnp.concatenate(
            [jnp.sum(h, 0, keepdims=True),
             jnp.sum(h * h, 0, keepdims=True)], 0)

        @pl.when(i == 0)
        def _():
            st_ref[...] = st

        @pl.when(i > 0)
        def _():
            st_ref[...] += st

    grid = N // B
    return pl.pallas_call(
        body,
        grid=(grid,),
        in_specs=[
            pl.BlockSpec((NC, B, F), lambda i: (0, i, 0)),
            pl.BlockSpec((B, F), lambda i: (i, 0)),
            pl.BlockSpec((B, 1), lambda i: (i, 0)),
            pl.BlockSpec((1, F), lambda i: (0, 0)),
        ],
        out_specs=[
            pl.BlockSpec((B, F), lambda i: (i, 0)),
            pl.BlockSpec((2, F), lambda i: (0, 0)),
        ],
        out_shape=[
            jax.ShapeDtypeStruct((N, F), jnp.float32),
            jax.ShapeDtypeStruct((2, F), jnp.float32),
        ],
    )(part, xs, dinv, b)


def _tc_bn_apply(h3, stats, gamma, beta, dinv, W, B=2000):
    F = h3.shape[1]
    Fo = W.shape[1]

    def body(h_ref, st_ref, g_ref, be_ref, d_ref, w_ref, o_ref):
        mean = st_ref[0:1] * (1.0 / N)
        var = st_ref[1:2] * (1.0 / N) - mean * mean
        y = (h_ref[...] - mean) * lax.rsqrt(var + 1e-5) * g_ref[...] + be_ref[...]
        o_ref[...] = d_ref[...] * jnp.dot(y, w_ref[...],
                                          preferred_element_type=jnp.float32)

    grid = N // B
    return pl.pallas_call(
        body,
        grid=(grid,),
        in_specs=[
            pl.BlockSpec((B, F), lambda i: (i, 0)),
            pl.BlockSpec((2, F), lambda i: (0, 0)),
            pl.BlockSpec((1, F), lambda i: (0, 0)),
            pl.BlockSpec((1, F), lambda i: (0, 0)),
            pl.BlockSpec((B, 1), lambda i: (i, 0)),
            pl.BlockSpec(W.shape, lambda i: (0, 0)),
        ],
        out_specs=pl.BlockSpec((B, Fo), lambda i: (i, 0)),
        out_shape=jax.ShapeDtypeStruct((N, Fo), jnp.float32),
    )(h3, stats, gamma, beta, dinv, W)


def _tc_head(part, xs, dinv, b, fc1_Wt, fc1_b, fc2_Wt, fc2_b, fc3_Wt, fc3_b,
             B=2000):
    Fin = xs.shape[1]
    G = fc1_Wt.shape[1]
    H = fc2_Wt.shape[1]
    Fo = fc3_Wt.shape[1]
    NG_ = G // CLUST

    def body(p_ref, xs_ref, d_ref, b_ref, w1_ref, b1_ref, w2_ref, b2_ref,
             w3_ref, b3_ref, o_ref):
        bf = jnp.bfloat16
        z = d_ref[...] * (p_ref[0] + p_ref[1] + xs_ref[...]) + b_ref[...]
        t = _elu(jnp.dot(z.astype(bf), w1_ref[...].astype(bf),
                         preferred_element_type=jnp.float32) + b1_ref[...])
        w2 = w2_ref[...].astype(bf)
        w3 = w3_ref[...].astype(bf)
        b2 = b2_ref[...]
        b3 = b3_ref[...]
        for j in range(NG_):
            u = jnp.dot(t[:, j * CLUST:(j + 1) * CLUST].astype(bf), w2,
                        preferred_element_type=jnp.float32) + b2
            o_ref[:, j * Fo:(j + 1) * Fo] = jnp.dot(
                u.astype(bf), w3, preferred_element_type=jnp.float32) + b3

    grid = N // B
    return pl.pallas_call(
        body,
        grid=(grid,),
        in_specs=[
            pl.BlockSpec((NC, B, Fin), lambda i: (0, i, 0)),
            pl.BlockSpec((B, Fin), lambda i: (i, 0)),
            pl.BlockSpec((B, 1), lambda i: (i, 0)),
            pl.BlockSpec((1, Fin), lambda i: (0, 0)),
            pl.BlockSpec(fc1_Wt.shape, lambda i: (0, 0)),
            pl.BlockSpec((1, G), lambda i: (0, 0)),
            pl.BlockSpec(fc2_Wt.shape, lambda i: (0, 0)),
            pl.BlockSpec((1, H), lambda i: (0, 0)),
            pl.BlockSpec(fc3_Wt.shape, lambda i: (0, 0)),
            pl.BlockSpec((1, Fo), lambda i: (0, 0)),
        ],
        out_specs=pl.BlockSpec((B, NG_ * Fo), lambda i: (i, 0)),
        out_shape=jax.ShapeDtypeStruct((N, NG_ * Fo), jnp.float32),
    )(part, xs, dinv, b, fc1_Wt, fc1_b, fc2_Wt, fc2_b, fc3_Wt, fc3_b)


def kernel(x, adj, num_graphs, in_batch, cluster, W1, b1, W2, b2, W3, b3,
           bn_gamma, bn_beta, W4, b4, W5, b5, fc1_W, fc1_b, fc2_W, fc2_b,
           fc3_W, fc3_b):
    if EPAD == E:
        src3 = adj[0].reshape(NW, NB, K)
        dst3 = adj[1].reshape(NW, NB, K)
    else:
        ar = jnp.arange(EPAD - E, dtype=jnp.int32)
        src3 = jnp.concatenate([adj[0], ar % N]).reshape(NW, NB, K)
        dst3 = jnp.concatenate([adj[1], N + ar % (NPAD - N)]).reshape(NW, NB, K)
    zs128 = jnp.zeros((NPAD, 128), jnp.float32)

    W4p = jnp.pad(W4, ((0, 0), (0, 64)))
    b4p = jnp.pad(b4, (0, 64))
    W5p = jnp.pad(W5, ((0, 64), (0, 64)))
    b5p = jnp.pad(b5, (0, 64))
    fc1_Wtp = jnp.pad(fc1_W.T, ((0, 64), (0, 0)))

    degp = _sc_degree(dst3).reshape(NC, NPAD, 1)

    xs1, dinv = _tc_first(x, W1, degp)
    p1 = _sc_aggregate(xs1, src3, dst3, zs128, 128)
    xs2 = _tc_mid(p1, xs1, dinv, b1.reshape(1, -1), W2, True)
    p2 = _sc_aggregate(xs2, src3, dst3, zs128, 128)
    xs3 = _tc_mid(p2, xs2, dinv, b2.reshape(1, -1), W3, True)
    p3 = _sc_aggregate(xs3, src3, dst3, zs128, 128)
    h3, stats = _tc_bn_stats(p3, xs3, dinv, b3.reshape(1, -1))
    xs4 = _tc_bn_apply(h3, stats, bn_gamma.reshape(1, -1),
                       bn_beta.reshape(1, -1), dinv, W4p)
    p4 = _sc_aggregate(xs4, src3, dst3, zs128, 128)
    xs5 = _tc_mid(p4, xs4, dinv, b4p.reshape(1, -1), W5p, False)
    p5 = _sc_aggregate(xs5, src3, dst3, zs128, 128)
    out100 = _tc_head(p5, xs5, dinv, b5p.reshape(1, -1), fc1_Wtp,
                      fc1_b.reshape(1, -1), fc2_W.T, fc2_b.reshape(1, -1),
                      fc3_W.T, fc3_b.reshape(1, -1))
    return out100.reshape(-1, 10)


# speedup vs baseline: 1.0274x; 1.0054x over previous
import functools

import jax
import jax.numpy as jnp
from jax import lax
from jax.experimental import pallas as pl
from jax.experimental.pallas import tpu as pltpu
from jax.experimental.pallas import tpu_sc as plsc

N = 10000
E = 320000
CLUST = 100

NC = 2
NS = 16
NW = NC * NS
K = 80
NB = 125
EPAD = NW * NB * K
ROWS_PT = 640
NPAD = NS * ROWS_PT

_SC_MESH = plsc.VectorSubcoreMesh(
    core_axis_name="c", subcore_axis_name="s", num_cores=NC, num_subcores=NS)


def _sc_degree(dst3):
    @functools.partial(
        pl.kernel,
        out_type=jax.ShapeDtypeStruct((NC, NPAD), jnp.float32),
        mesh=_SC_MESH,
        scratch_types=[
            pltpu.VMEM((NB, K), jnp.int32),
            pltpu.VMEM((K,), jnp.float32),
            pltpu.VMEM((ROWS_PT,), jnp.float32),
            pltpu.VMEM_SHARED((NPAD,), jnp.float32),
        ],
    )
    def deg_kernel(dst_hbm, out_hbm, dst_v, ones_v, z_v, acc):
        c = lax.axis_index("c")
        s = lax.axis_index("s")
        wid = c * NS + s
        r0 = s * ROWS_PT
        for k in range(K // 16):
            ones_v[pl.ds(k * 16, 16)] = jnp.ones((16,), jnp.float32)
        for k in range(ROWS_PT // 16):
            z_v[pl.ds(k * 16, 16)] = jnp.zeros((16,), jnp.float32)
        pltpu.sync_copy(z_v, acc.at[pl.ds(r0, ROWS_PT)])
        pltpu.sync_copy(dst_hbm.at[wid], dst_v)
        plsc.subcore_barrier()

        def body(j, carry):
            pltpu.sync_copy(ones_v, acc.at[dst_v.at[j]], add=True)
            return carry

        lax.fori_loop(0, NB, body, 0)
        plsc.subcore_barrier()
        pltpu.sync_copy(acc.at[pl.ds(r0, ROWS_PT)],
                        out_hbm.at[c, pl.ds(r0, ROWS_PT)])

    return deg_kernel(dst3)


def _sc_aggregate(xs, src3, dst3, zs, F):
    @functools.partial(
        pl.kernel,
        out_type=jax.ShapeDtypeStruct((NC, NPAD, F), jnp.float32),
        mesh=_SC_MESH,
        scratch_types=[
            pltpu.VMEM((4, K), jnp.int32),
            pltpu.VMEM((4, K), jnp.int32),
            pltpu.VMEM((3, K, F), jnp.float32),
            pltpu.VMEM_SHARED((NPAD, F), jnp.float32),
            pltpu.SemaphoreType.DMA((2,)),
            pltpu.SemaphoreType.DMA((2,)),
            pltpu.SemaphoreType.DMA,
        ],
    )
    def agg_kernel(xs_hbm, src_hbm, dst_hbm, zs_hbm, out_hbm,
                   isrc, idst, rows_v, acc, isem, gsem, ssem):
        c = lax.axis_index("c")
        s = lax.axis_index("s")
        wid = c * NS + s
        r0 = s * ROWS_PT
        pltpu.sync_copy(zs_hbm.at[pl.ds(r0, ROWS_PT)],
                        acc.at[pl.ds(r0, ROWS_PT)])
        pltpu.sync_copy(src_hbm.at[wid, 0], isrc.at[0])
        pltpu.sync_copy(dst_hbm.at[wid, 0], idst.at[0])
        pltpu.sync_copy(src_hbm.at[wid, 1], isrc.at[1])
        pltpu.sync_copy(dst_hbm.at[wid, 1], idst.at[1])
        pltpu.async_copy(src_hbm.at[wid, 2], isrc.at[2], isem.at[0])
        pltpu.async_copy(dst_hbm.at[wid, 2], idst.at[2], isem.at[0])
        plsc.subcore_barrier()
        pltpu.async_copy(xs_hbm.at[isrc.at[0]], rows_v.at[0], gsem.at[0])
        pltpu.async_copy(xs_hbm.at[isrc.at[1]], rows_v.at[1], gsem.at[1])

        def body(j, carry):
            m = lax.rem(j, 3)
            m2 = lax.rem(j + 2, 3)
            hj = lax.rem(j, 2)
            h2 = hj
            i_cur = lax.rem(j, 4)
            i_nxt2 = lax.rem(j + 2, 4)
            i_nxt3 = lax.rem(j + 3, 4)
            pltpu.make_async_copy(xs_hbm.at[isrc.at[i_cur]],
                                  rows_v.at[m], gsem.at[hj]).wait()

            @pl.when(j >= 1)
            def _():
                pltpu.make_async_copy(rows_v.at[m2],
                                      acc.at[idst.at[i_nxt3]], ssem).wait()

            @pl.when(j + 3 < NB)
            def _():
                pltpu.async_copy(src_hbm.at[wid, j + 3], isrc.at[i_nxt3],
                                 isem.at[1 - hj])
                pltpu.async_copy(dst_hbm.at[wid, j + 3], idst.at[i_nxt3],
                                 isem.at[1 - hj])

            @pl.when(j + 2 < NB)
            def _():
                pltpu.make_async_copy(src_hbm.at[wid, j + 2],
                                      isrc.at[i_nxt2], isem.at[h2]).wait()
                pltpu.make_async_copy(dst_hbm.at[wid, j + 2],
                                      idst.at[i_nxt2], isem.at[h2]).wait()
                pltpu.async_copy(xs_hbm.at[isrc.at[i_nxt2]],
                                 rows_v.at[m2], gsem.at[h2])

            pltpu.async_copy(rows_v.at[m], acc.at[idst.at[i_cur]],
                             ssem, add=True)
            return carry

        lax.fori_loop(0, NB, body, 0)
        pltpu.make_async_copy(rows_v.at[(NB - 1) % 3],
                              acc.at[idst.at[(NB - 1) % 4]], ssem).wait()
        plsc.subcore_barrier()
        pltpu.sync_copy(acc.at[pl.ds(r0, ROWS_PT)],
                        out_hbm.at[c, pl.ds(r0, ROWS_PT)])

    return agg_kernel(xs, src3, dst3, zs)


def _elu(v):
    return jnp.where(v > 0, v, jnp.exp(jnp.minimum(v, 0.0)) - 1.0)


def _tc_first(x, W, degp, B=2000):
    F = W.shape[1]

    def body(x_ref, w_ref, d_ref, xs_ref, dinv_ref):
        deg = d_ref[0] + d_ref[1] + 1.0
        dinv = lax.rsqrt(deg)
        dinv_ref[...] = dinv
        xs_ref[...] = dinv * jnp.dot(x_ref[...], w_ref[...],
                                     preferred_element_type=jnp.float32)

    grid = N // B
    return pl.pallas_call(
        body,
        grid=(grid,),
        in_specs=[
            pl.BlockSpec((B, x.shape[1]), lambda i: (i, 0)),
            pl.BlockSpec(W.shape, lambda i: (0, 0)),
            pl.BlockSpec((NC, B, 1), lambda i: (0, i, 0)),
        ],
        out_specs=[
            pl.BlockSpec((B, F), lambda i: (i, 0)),
            pl.BlockSpec((B, 1), lambda i: (i, 0)),
        ],
        out_shape=[
            jax.ShapeDtypeStruct((N, F), jnp.float32),
            jax.ShapeDtypeStruct((N, 1), jnp.float32),
        ],
    )(x, W, degp)


def _tc_mid(part, xs, dinv, b, W, apply_elu, B=2000):
    Fin = xs.shape[1]
    F = W.shape[1]

    def body(p_ref, xs_ref, d_ref, b_ref, w_ref, o_ref):
        dinv = d_ref[...]
        h = dinv * (p_ref[0] + p_ref[1] + xs_ref[...]) + b_ref[...]
        if apply_elu:
            h = _elu(h)
        o_ref[...] = dinv * jnp.dot(h, w_ref[...],
                                    preferred_element_type=jnp.float32)

    grid = N // B
    return pl.pallas_call(
        body,
        grid=(grid,),
        in_specs=[
            pl.BlockSpec((NC, B, Fin), lambda i: (0, i, 0)),
            pl.BlockSpec((B, Fin), lambda i: (i, 0)),
            pl.BlockSpec((B, 1), lambda i: (i, 0)),
            pl.BlockSpec((1, Fin), lambda i: (0, 0)),
            pl.BlockSpec(W.shape, lambda i: (0, 0)),
        ],
        out_specs=pl.BlockSpec((B, F), lambda i: (i, 0)),
        out_shape=jax.ShapeDtypeStruct((N, F), jnp.float32),
    )(part, xs, dinv, b, W)


def _tc_bn(part, xs, dinv, b, gamma, beta, W, B=2000):
    F = xs.shape[1]
    Fo = W.shape[1]
    grid = N // B

    def body(p_ref, xs_ref, d_ref, b_ref, g_ref, be_ref, w_ref, o_ref,
             h_scr, st_scr):
        i = pl.program_id(0)

        @pl.when(i < grid)
        def _():
            h = d_ref[...] * (p_ref[0] + p_ref[1] + xs_ref[...]) + b_ref[...]
            h_scr[i] = h
            st = jnp.concatenate(
                [jnp.sum(h, 0, keepdims=True),
                 jnp.sum(h * h, 0, keepdims=True)], 0)

            @pl.when(i == 0)
            def _():
                st_scr[...] = st

            @pl.when(i > 0)
            def _():
                st_scr[...] += st

        @pl.when(i >= grid)
        def _():
            mean = st_scr[0:1] * (1.0 / N)
            var = st_scr[1:2] * (1.0 / N) - mean * mean
            y = ((h_scr[i - grid] - mean) * lax.rsqrt(var + 1e-5)
                 * g_ref[...] + be_ref[...])
            o_ref[...] = d_ref[...] * jnp.dot(
                y, w_ref[...], preferred_element_type=jnp.float32)

    def rowmap(i):
        return (lax.rem(i, grid), 0)

    def clampmap(i):
        return (lax.min(i, grid - 1), 0)

    return pl.pallas_call(
        body,
        grid=(2 * grid,),
        in_specs=[
            pl.BlockSpec((NC, B, F),
                         lambda i: (0, lax.min(i, grid - 1), 0)),
            pl.BlockSpec((B, F), clampmap),
            pl.BlockSpec((B, 1), rowmap),
            pl.BlockSpec((1, F), lambda i: (0, 0)),
            pl.BlockSpec((1, F), lambda i: (0, 0)),
            pl.BlockSpec((1, F), lambda i: (0, 0)),
            pl.BlockSpec(W.shape, lambda i: (0, 0)),
        ],
        out_specs=pl.BlockSpec((B, Fo), rowmap),
        out_shape=jax.ShapeDtypeStruct((N, Fo), jnp.float32),
        scratch_shapes=[
            pltpu.VMEM((grid, B, F), jnp.float32),
            pltpu.VMEM((2, F), jnp.float32),
        ],
    )(part, xs, dinv, b, gamma, beta, W)


def _tc_head(part, xs, dinv, b, fc1_Wt, fc1_b, fc2_Wt, fc2_b, fc3_Wt, fc3_b,
             B=2000):
    Fin = xs.shape[1]
    G = fc1_Wt.shape[1]
    H = fc2_Wt.shape[1]
    Fo = fc3_Wt.shape[1]
    NG_ = G // CLUST

    def body(p_ref, xs_ref, d_ref, b_ref, w1_ref, b1_ref, w2_ref, b2_ref,
             w3_ref, b3_ref, o_ref):
        bf = jnp.bfloat16
        z = d_ref[...] * (p_ref[0] + p_ref[1] + xs_ref[...]) + b_ref[...]
        t = _elu(jnp.dot(z.astype(bf), w1_ref[...].astype(bf),
                         preferred_element_type=jnp.float32) + b1_ref[...])
        w2 = w2_ref[...].astype(bf)
        w3 = w3_ref[...].astype(bf)
        b2 = b2_ref[...]
        b3 = b3_ref[...]
        for j in range(NG_):
            u = jnp.dot(t[:, j * CLUST:(j + 1) * CLUST].astype(bf), w2,
                        preferred_element_type=jnp.float32) + b2
            o_ref[:, j * Fo:(j + 1) * Fo] = jnp.dot(
                u.astype(bf), w3, preferred_element_type=jnp.float32) + b3

    grid = N // B
    return pl.pallas_call(
        body,
        grid=(grid,),
        in_specs=[
            pl.BlockSpec((NC, B, Fin), lambda i: (0, i, 0)),
            pl.BlockSpec((B, Fin), lambda i: (i, 0)),
            pl.BlockSpec((B, 1), lambda i: (i, 0)),
            pl.BlockSpec((1, Fin), lambda i: (0, 0)),
            pl.BlockSpec(fc1_Wt.shape, lambda i: (0, 0)),
            pl.BlockSpec((1, G), lambda i: (0, 0)),
            pl.BlockSpec(fc2_Wt.shape, lambda i: (0, 0)),
            pl.BlockSpec((1, H), lambda i: (0, 0)),
            pl.BlockSpec(fc3_Wt.shape, lambda i: (0, 0)),
            pl.BlockSpec((1, Fo), lambda i: (0, 0)),
        ],
        out_specs=pl.BlockSpec((B, NG_ * Fo), lambda i: (i, 0)),
        out_shape=jax.ShapeDtypeStruct((N, NG_ * Fo), jnp.float32),
    )(part, xs, dinv, b, fc1_Wt, fc1_b, fc2_Wt, fc2_b, fc3_Wt, fc3_b)


def kernel(x, adj, num_graphs, in_batch, cluster, W1, b1, W2, b2, W3, b3,
           bn_gamma, bn_beta, W4, b4, W5, b5, fc1_W, fc1_b, fc2_W, fc2_b,
           fc3_W, fc3_b):
    if EPAD == E:
        src3 = adj[0].reshape(NW, NB, K)
        dst3 = adj[1].reshape(NW, NB, K)
    else:
        ar = jnp.arange(EPAD - E, dtype=jnp.int32)
        src3 = jnp.concatenate([adj[0], ar % N]).reshape(NW, NB, K)
        dst3 = jnp.concatenate([adj[1], N + ar % (NPAD - N)]).reshape(NW, NB, K)
    zs128 = jnp.zeros((NPAD, 128), jnp.float32)

    W4p = jnp.pad(W4, ((0, 0), (0, 64)))
    b4p = jnp.pad(b4, (0, 64))
    W5p = jnp.pad(W5, ((0, 64), (0, 64)))
    b5p = jnp.pad(b5, (0, 64))
    fc1_Wtp = jnp.pad(fc1_W.T, ((0, 64), (0, 0)))

    degp = _sc_degree(dst3).reshape(NC, NPAD, 1)

    xs1, dinv = _tc_first(x, W1, degp)
    p1 = _sc_aggregate(xs1, src3, dst3, zs128, 128)
    xs2 = _tc_mid(p1, xs1, dinv, b1.reshape(1, -1), W2, True)
    p2 = _sc_aggregate(xs2, src3, dst3, zs128, 128)
    xs3 = _tc_mid(p2, xs2, dinv, b2.reshape(1, -1), W3, True)
    p3 = _sc_aggregate(xs3, src3, dst3, zs128, 128)
    xs4 = _tc_bn(p3, xs3, dinv, b3.reshape(1, -1), bn_gamma.reshape(1, -1),
                 bn_beta.reshape(1, -1), W4p)
    p4 = _sc_aggregate(xs4, src3, dst3, zs128, 128)
    xs5 = _tc_mid(p4, xs4, dinv, b4p.reshape(1, -1), W5p, False)
    p5 = _sc_aggregate(xs5, src3, dst3, zs128, 128)
    out100 = _tc_head(p5, xs5, dinv, b5p.reshape(1, -1), fc1_Wtp,
                      fc1_b.reshape(1, -1), fc2_W.T, fc2_b.reshape(1, -1),
                      fc3_W.T, fc3_b.reshape(1, -1))
    return out100.reshape(-1, 10)
